# baseline (speedup 1.0000x reference)
import functools

import jax
import jax.numpy as jnp
from jax import lax
from jax.experimental import pallas as pl
from jax.experimental.pallas import tpu as pltpu
from jax.experimental.pallas import tpu_sc as plsc

_NC, _NS, _L = 2, 16, 16
_NW = _NC * _NS


def _argmax_kernel(z_ref, cb_ref, kr_ref, g_ref, m_ref):
    zb = z_ref[...]
    cb3 = cb_ref[...].reshape(1, 1, -1)
    k = cb3.shape[-1]
    krev3 = kr_ref[...].reshape(1, 1, k)
    y = g_ref[...] - jnp.abs(zb[:, :, None] - cb3)
    v = jnp.max(y, axis=-1, keepdims=True)
    cand = jnp.where(y == v, krev3, jnp.float32(0.0))
    m_ref[...] = (jnp.float32(k - 1) - jnp.max(cand, axis=-1)
                  ).astype(jnp.int32)


def _sc_argmax_kernel(g_hbm, z_hbm, cb_hbm, mt_hbm,
                      z_v, cb_v, m_v, gb0, gb1, sem0, sem1,
                      *, kk, base_elem, rpt):
    wid = lax.axis_index("s") * _NC + lax.axis_index("c")
    base = base_elem + wid * rpt
    pltpu.sync_copy(z_hbm.at[pl.ds(base, rpt)], z_v)
    pltpu.sync_copy(cb_hbm, cb_v)

    gr = _L
    ng = rpt // gr
    gsz = gr * kk

    def gsrc(g):
        return g_hbm.at[pl.ds((base + g * gr) * kk, gsz)]

    pltpu.async_copy(gsrc(0), gb0, sem0)

    roff = lax.broadcasted_iota(jnp.int32, (_L,), 0) * kk
    neg = jnp.full((_L,), -3.4e38, jnp.float32)
    unroll = 8

    def compute(g, buf):
        z16 = z_v[pl.ds(g * gr, gr)]

        def kbody(kb, carry):
            best, bidx = carry
            for u in range(unroll):
                k = kb * unroll + u
                kf = jnp.full((_L,), k, jnp.int32)
                gv = plsc.load_gather(buf, [roff + k])
                ck = plsc.load_gather(cb_v, [kf])
                y = gv - jnp.abs(z16 - ck)
                upd = y > best
                best = jnp.maximum(best, y)
                bidx = jnp.where(upd, kf, bidx)
            return best, bidx

        _, bidx = lax.fori_loop(0, kk // unroll, kbody,
                                (neg, jnp.zeros((_L,), jnp.int32)))
        m_v[pl.ds(g * gr, gr)] = bidx

    def loop(j, carry):
        ge = j * 2
        pltpu.async_copy(gsrc(ge + 1), gb1, sem1)
        pltpu.make_async_copy(gsrc(ge), gb0, sem0).wait()
        compute(ge, gb0)

        @pl.when(ge + 2 < ng)
        def _():
            pltpu.async_copy(gsrc(ge + 2), gb0, sem0)

        pltpu.make_async_copy(gsrc(ge + 1), gb1, sem1).wait()
        compute(ge + 1, gb1)
        return carry

    lax.fori_loop(0, ng // 2, loop, 0)
    pltpu.sync_copy(m_v, mt_hbm.at[pl.ds(wid * rpt, rpt)])


def _sc_kernel(m_hbm, z_hbm, cb_hbm, disc_hbm, enc_hbm, hist_hbm,
               m_v, z_v, cb_v, disc_v, enc_v, histf_v, histo_v,
               *, chunk, iters, kk):
    wid = lax.axis_index("s") * _NC + lax.axis_index("c")
    base = wid * chunk
    pltpu.sync_copy(m_hbm.at[pl.ds(base, chunk)], m_v)
    pltpu.sync_copy(z_hbm.at[pl.ds(base, chunk)], z_v)
    pltpu.sync_copy(cb_hbm, cb_v)

    zeros16 = jnp.zeros((_L,), jnp.float32)
    for j in range(_L * kk // _L):
        histf_v[pl.ds(j * _L, _L)] = zeros16
    ones16 = jnp.ones((_L,), jnp.float32)
    laneoff = lax.broadcasted_iota(jnp.int32, (_L,), 0) * kk
    kmax = kk - 1

    def body(i, carry):
        off = i * _L
        mv = m_v[pl.ds(off, _L)]
        zv = z_v[pl.ds(off, _L)]
        disc_v[pl.ds(off, _L)] = plsc.load_gather(cb_v, [mv])
        plsc.addupdate_scatter(histf_v, [laneoff + mv], ones16)

        x = (zv + 1.0) * (kmax / 2.0)
        x = jnp.minimum(jnp.maximum(x, 0.0), float(kmax))
        e0 = (x + 0.5).astype(jnp.int32)
        a = jnp.maximum(e0 - 1, 0)
        b = jnp.minimum(e0, kmax)
        c = jnp.minimum(e0 + 1, kmax)
        da = jnp.abs(zv - plsc.load_gather(cb_v, [a]))
        db = jnp.abs(zv - plsc.load_gather(cb_v, [b]))
        dc = jnp.abs(zv - plsc.load_gather(cb_v, [c]))
        bi = a
        bd = da
        upd = db < bd
        bi = jnp.where(upd, b, bi)
        bd = jnp.where(upd, db, bd)
        bi = jnp.where(dc < bd, c, bi)
        enc_v[pl.ds(off, _L)] = bi
        return carry

    lax.fori_loop(0, iters, body, 0)

    for cidx in range(kk // _L):
        acc = histf_v[pl.ds(cidx * _L, _L)]
        for l in range(1, _L):
            acc = acc + histf_v[pl.ds(l * kk + cidx * _L, _L)]
        histo_v[pl.ds(cidx * _L, _L)] = acc

    pltpu.sync_copy(disc_v, disc_hbm.at[pl.ds(base, chunk)])
    pltpu.sync_copy(enc_v, enc_hbm.at[pl.ds(base, chunk)])
    pltpu.sync_copy(histo_v, hist_hbm.at[wid])


def _perp_kernel(h_ref, p_ref, *, n_total):
    avg = jnp.sum(h_ref[...], axis=0) / jnp.float32(n_total)
    p_ref[...] = jnp.exp(-jnp.sum(avg * jnp.log(avg + 1e-10))).reshape(1, 1)


def kernel(z, codebook, log_temperature, gumbel_noise):
    B, T, D = z.shape
    K = codebook.shape[0]
    N = B * T * D
    NR = N // K
    R = 56
    SROWS = 112
    TROWS = NR - SROWS
    nsteps = pl.cdiv(TROWS, R)

    z2 = z.reshape(NR, K)
    g3 = gumbel_noise.reshape(NR, K, K)
    cb2 = codebook.reshape(1, K)
    krev2 = jnp.arange(K - 1, -1, -1, dtype=jnp.float32).reshape(1, K)

    m_head = pl.pallas_call(
        _argmax_kernel,
        grid=(nsteps,),
        in_specs=[
            pl.BlockSpec((R, K), lambda b: (b, 0)),
            pl.BlockSpec((1, K), lambda b: (0, 0)),
            pl.BlockSpec((1, K), lambda b: (0, 0)),
            pl.BlockSpec((R, K, K), lambda b: (b, 0, 0)),
        ],
        out_specs=pl.BlockSpec((R, K), lambda b: (b, 0)),
        out_shape=jax.ShapeDtypeStruct((TROWS, K), jnp.int32),
    )(z2, cb2, krev2, g3)

    rpt = SROWS * K // _NW
    m_tail = pl.kernel(
        functools.partial(_sc_argmax_kernel, kk=K, base_elem=TROWS * K,
                          rpt=rpt),
        out_type=jax.ShapeDtypeStruct((SROWS * K,), jnp.int32),
        mesh=plsc.VectorSubcoreMesh(core_axis_name="c", subcore_axis_name="s",
                                    num_cores=_NC, num_subcores=_NS),
        compiler_params=pltpu.CompilerParams(needs_layout_passes=False),
        scratch_types=[
            pltpu.VMEM((rpt,), jnp.float32),
            pltpu.VMEM((K,), jnp.float32),
            pltpu.VMEM((rpt,), jnp.int32),
            pltpu.VMEM((_L * K,), jnp.float32),
            pltpu.VMEM((_L * K,), jnp.float32),
            pltpu.SemaphoreType.DMA,
            pltpu.SemaphoreType.DMA,
        ],
    )(gumbel_noise.reshape(-1), z.reshape(N), codebook)

    m = jnp.concatenate([m_head.reshape(-1), m_tail])

    chunk = N // _NW
    disc, enc, hist = pl.kernel(
        functools.partial(_sc_kernel, chunk=chunk, iters=chunk // _L, kk=K),
        out_type=[
            jax.ShapeDtypeStruct((N,), jnp.float32),
            jax.ShapeDtypeStruct((N,), jnp.int32),
            jax.ShapeDtypeStruct((_NW, K), jnp.float32),
        ],
        mesh=plsc.VectorSubcoreMesh(core_axis_name="c", subcore_axis_name="s",
                                    num_cores=_NC, num_subcores=_NS),
        compiler_params=pltpu.CompilerParams(needs_layout_passes=False),
        scratch_types=[
            pltpu.VMEM((chunk,), jnp.int32),
            pltpu.VMEM((chunk,), jnp.float32),
            pltpu.VMEM((K,), jnp.float32),
            pltpu.VMEM((chunk,), jnp.float32),
            pltpu.VMEM((chunk,), jnp.int32),
            pltpu.VMEM((_L * K,), jnp.float32),
            pltpu.VMEM((K,), jnp.float32),
        ],
    )(m, z.reshape(N), codebook)

    perp = pl.pallas_call(
        functools.partial(_perp_kernel, n_total=N),
        in_specs=[pl.BlockSpec((_NW, K), lambda: (0, 0))],
        out_specs=pl.BlockSpec((1, 1), lambda: (0, 0)),
        out_shape=jax.ShapeDtypeStruct((1, 1), jnp.float32),
    )(hist)

    return (disc.reshape(B, T, D), perp[0, 0], enc)


# speedup vs baseline: 1.0658x; 1.0658x over previous
import functools

import jax
import jax.numpy as jnp
from jax import lax
from jax.experimental import pallas as pl
from jax.experimental.pallas import tpu as pltpu
from jax.experimental.pallas import tpu_sc as plsc

_NC, _NS, _L = 2, 16, 16
_NW = _NC * _NS


def _argmax_kernel(z_ref, cb_ref, kr_ref, g_ref, m_ref):
    zb = z_ref[...]
    cb3 = cb_ref[...].reshape(1, 1, -1)
    k = cb3.shape[-1]
    krev3 = kr_ref[...].reshape(1, 1, k)
    y = g_ref[...] - jnp.abs(zb[:, :, None] - cb3)
    v = jnp.max(y, axis=-1, keepdims=True)
    cand = jnp.where(y == v, krev3, jnp.float32(0.0))
    m_ref[...] = (jnp.float32(k - 1) - jnp.max(cand, axis=-1)
                  ).astype(jnp.int32)


def _sc_argmax_kernel(g_hbm, z_hbm, cb_hbm, mt_hbm,
                      z_v, cb_v, m_v, gb0, gb1, sem0, sem1,
                      *, kk, base_elem, rpt):
    wid = lax.axis_index("s") * _NC + lax.axis_index("c")
    base = base_elem + wid * rpt
    pltpu.sync_copy(z_hbm.at[pl.ds(base, rpt)], z_v)
    pltpu.sync_copy(cb_hbm, cb_v)

    gr = _L
    ng = rpt // gr
    gsz = gr * kk

    def gsrc(g):
        return g_hbm.at[pl.ds((base + g * gr) * kk, gsz)]

    pltpu.async_copy(gsrc(0), gb0, sem0)

    roff = lax.broadcasted_iota(jnp.int32, (_L,), 0) * kk
    neg = jnp.full((_L,), -3.4e38, jnp.float32)
    zero_i = jnp.zeros((_L,), jnp.int32)
    unroll, nacc = 8, 4

    def compute(g, buf):
        z16 = z_v[pl.ds(g * gr, gr)]

        def kbody(kb, carry):
            accs = list(carry)
            for u in range(unroll):
                k = kb * unroll + u
                a = u % nacc
                best, bidx = accs[a]
                kf = jnp.full((_L,), k, jnp.int32)
                gv = plsc.load_gather(buf, [roff + k])
                ck = plsc.load_gather(cb_v, [kf])
                y = gv - jnp.abs(z16 - ck)
                upd = y > best
                best = jnp.maximum(best, y)
                bidx = jnp.where(upd, kf, bidx)
                accs[a] = (best, bidx)
            return tuple(accs)

        accs = lax.fori_loop(0, kk // unroll, kbody,
                             tuple((neg, zero_i) for _ in range(nacc)))
        best, bidx = accs[0]
        for b2, i2 in accs[1:]:
            take = (b2 > best) | ((b2 == best) & (i2 < bidx))
            best = jnp.where(take, b2, best)
            bidx = jnp.where(take, i2, bidx)
        m_v[pl.ds(g * gr, gr)] = bidx

    def loop(j, carry):
        ge = j * 2
        pltpu.async_copy(gsrc(ge + 1), gb1, sem1)
        pltpu.make_async_copy(gsrc(ge), gb0, sem0).wait()
        compute(ge, gb0)

        @pl.when(ge + 2 < ng)
        def _():
            pltpu.async_copy(gsrc(ge + 2), gb0, sem0)

        pltpu.make_async_copy(gsrc(ge + 1), gb1, sem1).wait()
        compute(ge + 1, gb1)
        return carry

    lax.fori_loop(0, ng // 2, loop, 0)
    if ng % 2 == 1:
        pltpu.make_async_copy(gsrc(ng - 1), gb0, sem0).wait()
        compute(ng - 1, gb0)
    pltpu.sync_copy(m_v, mt_hbm.at[pl.ds(wid * rpt, rpt)])


def _sc_kernel(mh_hbm, mt_hbm, z_hbm, cb_hbm, disc_hbm, enc_hbm, hist_hbm,
               m_v, z_v, cb_v, disc_v, enc_v, histf_v, histo_v,
               *, chunk, iters, kk, telems):
    wid = lax.axis_index("s") * _NC + lax.axis_index("c")
    base = wid * chunk
    wsplit = telems // chunk

    @pl.when(wid < wsplit)
    def _():
        pltpu.sync_copy(mh_hbm.at[pl.ds(base, chunk)], m_v)

    @pl.when(wid >= wsplit)
    def _():
        pltpu.sync_copy(mt_hbm.at[pl.ds(base - telems, chunk)], m_v)

    pltpu.sync_copy(z_hbm.at[pl.ds(base, chunk)], z_v)
    pltpu.sync_copy(cb_hbm, cb_v)

    zeros16 = jnp.zeros((_L,), jnp.float32)
    for j in range(_L * kk // _L):
        histf_v[pl.ds(j * _L, _L)] = zeros16
    ones16 = jnp.ones((_L,), jnp.float32)
    laneoff = lax.broadcasted_iota(jnp.int32, (_L,), 0) * kk
    kmax = kk - 1

    def body(i, carry):
        off = i * _L
        mv = m_v[pl.ds(off, _L)]
        zv = z_v[pl.ds(off, _L)]
        disc_v[pl.ds(off, _L)] = plsc.load_gather(cb_v, [mv])
        plsc.addupdate_scatter(histf_v, [laneoff + mv], ones16)

        x = (zv + 1.0) * (kmax / 2.0)
        x = jnp.minimum(jnp.maximum(x, 0.0), float(kmax))
        e0 = (x + 0.5).astype(jnp.int32)
        a = jnp.maximum(e0 - 1, 0)
        b = jnp.minimum(e0, kmax)
        c = jnp.minimum(e0 + 1, kmax)
        da = jnp.abs(zv - plsc.load_gather(cb_v, [a]))
        db = jnp.abs(zv - plsc.load_gather(cb_v, [b]))
        dc = jnp.abs(zv - plsc.load_gather(cb_v, [c]))
        bi = a
        bd = da
        upd = db < bd
        bi = jnp.where(upd, b, bi)
        bd = jnp.where(upd, db, bd)
        bi = jnp.where(dc < bd, c, bi)
        enc_v[pl.ds(off, _L)] = bi
        return carry

    lax.fori_loop(0, iters, body, 0)

    for cidx in range(kk // _L):
        acc = histf_v[pl.ds(cidx * _L, _L)]
        for l in range(1, _L):
            acc = acc + histf_v[pl.ds(l * kk + cidx * _L, _L)]
        histo_v[pl.ds(cidx * _L, _L)] = acc

    pltpu.sync_copy(disc_v, disc_hbm.at[pl.ds(base, chunk)])
    pltpu.sync_copy(enc_v, enc_hbm.at[pl.ds(base, chunk)])
    pltpu.sync_copy(histo_v, hist_hbm.at[wid])


def _perp_kernel(h_ref, p_ref, *, n_total):
    avg = jnp.sum(h_ref[...], axis=0) / jnp.float32(n_total)
    p_ref[...] = jnp.exp(-jnp.sum(avg * jnp.log(avg + 1e-10))).reshape(1, 1)


def kernel(z, codebook, log_temperature, gumbel_noise):
    B, T, D = z.shape
    K = codebook.shape[0]
    N = B * T * D
    NR = N // K
    R = 56
    SROWS = 98
    TROWS = NR - SROWS
    nsteps = pl.cdiv(TROWS, R)

    z2 = z.reshape(NR, K)
    g3 = gumbel_noise.reshape(NR, K, K)
    cb2 = codebook.reshape(1, K)
    krev2 = jnp.arange(K - 1, -1, -1, dtype=jnp.float32).reshape(1, K)

    m_head = pl.pallas_call(
        _argmax_kernel,
        grid=(nsteps,),
        in_specs=[
            pl.BlockSpec((R, K), lambda b: (b, 0)),
            pl.BlockSpec((1, K), lambda b: (0, 0)),
            pl.BlockSpec((1, K), lambda b: (0, 0)),
            pl.BlockSpec((R, K, K), lambda b: (b, 0, 0)),
        ],
        out_specs=pl.BlockSpec((R, K), lambda b: (b, 0)),
        out_shape=jax.ShapeDtypeStruct((TROWS, K), jnp.int32),
    )(z2, cb2, krev2, g3)

    rpt = SROWS * K // _NW
    m_tail = pl.kernel(
        functools.partial(_sc_argmax_kernel, kk=K, base_elem=TROWS * K,
                          rpt=rpt),
        out_type=jax.ShapeDtypeStruct((SROWS * K,), jnp.int32),
        mesh=plsc.VectorSubcoreMesh(core_axis_name="c", subcore_axis_name="s",
                                    num_cores=_NC, num_subcores=_NS),
        compiler_params=pltpu.CompilerParams(needs_layout_passes=False),
        scratch_types=[
            pltpu.VMEM((rpt,), jnp.float32),
            pltpu.VMEM((K,), jnp.float32),
            pltpu.VMEM((rpt,), jnp.int32),
            pltpu.VMEM((_L * K,), jnp.float32),
            pltpu.VMEM((_L * K,), jnp.float32),
            pltpu.SemaphoreType.DMA,
            pltpu.SemaphoreType.DMA,
        ],
    )(gumbel_noise.reshape(-1), z.reshape(N), codebook)

    chunk = N // _NW
    disc, enc, hist = pl.kernel(
        functools.partial(_sc_kernel, chunk=chunk, iters=chunk // _L, kk=K,
                          telems=TROWS * K),
        out_type=[
            jax.ShapeDtypeStruct((N,), jnp.float32),
            jax.ShapeDtypeStruct((N,), jnp.int32),
            jax.ShapeDtypeStruct((_NW, K), jnp.float32),
        ],
        mesh=plsc.VectorSubcoreMesh(core_axis_name="c", subcore_axis_name="s",
                                    num_cores=_NC, num_subcores=_NS),
        compiler_params=pltpu.CompilerParams(needs_layout_passes=False),
        scratch_types=[
            pltpu.VMEM((chunk,), jnp.int32),
            pltpu.VMEM((chunk,), jnp.float32),
            pltpu.VMEM((K,), jnp.float32),
            pltpu.VMEM((chunk,), jnp.float32),
            pltpu.VMEM((chunk,), jnp.int32),
            pltpu.VMEM((_L * K,), jnp.float32),
            pltpu.VMEM((K,), jnp.float32),
        ],
    )(m_head.reshape(-1), m_tail, z.reshape(N), codebook)

    perp = pl.pallas_call(
        functools.partial(_perp_kernel, n_total=N),
        in_specs=[pl.BlockSpec((_NW, K), lambda: (0, 0))],
        out_specs=pl.BlockSpec((1, 1), lambda: (0, 0)),
        out_shape=jax.ShapeDtypeStruct((1, 1), jnp.float32),
    )(hist)

    return (disc.reshape(B, T, D), perp[0, 0], enc)


# speedup vs baseline: 2.8169x; 2.6430x over previous
import functools

import jax
import jax.numpy as jnp
from jax import lax
from jax.experimental import pallas as pl
from jax.experimental.pallas import tpu as pltpu
from jax.experimental.pallas import tpu_sc as plsc

_NC, _NS, _L = 2, 16, 16
_NW = _NC * _NS


def _argmax_kernel(z_ref, cb_ref, kr_ref, g_ref, m_ref):
    zb = z_ref[...]
    cb3 = cb_ref[...].reshape(1, 1, -1)
    k = cb3.shape[-1]
    krev3 = kr_ref[...].reshape(1, 1, k)
    y = g_ref[...] - jnp.abs(zb[:, :, None] - cb3)
    v = jnp.max(y, axis=-1, keepdims=True)
    cand = jnp.where(y == v, krev3, jnp.float32(0.0))
    m_ref[...] = (jnp.float32(k - 1) - jnp.max(cand, axis=-1)
                  ).astype(jnp.int32)


def _sc_kernel(m_hbm, z_hbm, cb_hbm, disc_hbm, enc_hbm, hist_hbm,
               m_v, z_v, cb_v, disc_v, enc_v, histf_v, histo_v,
               *, chunk, iters, kk):
    wid = lax.axis_index("s") * _NC + lax.axis_index("c")
    base = wid * chunk
    pltpu.sync_copy(m_hbm.at[pl.ds(base, chunk)], m_v)
    pltpu.sync_copy(z_hbm.at[pl.ds(base, chunk)], z_v)
    pltpu.sync_copy(cb_hbm, cb_v)

    zeros16 = jnp.zeros((_L,), jnp.float32)
    for j in range(_L * kk // _L):
        histf_v[pl.ds(j * _L, _L)] = zeros16
    ones16 = jnp.ones((_L,), jnp.float32)
    laneoff = lax.broadcasted_iota(jnp.int32, (_L,), 0) * kk
    kmax = kk - 1

    def body(i, carry):
        off = i * _L
        mv = m_v[pl.ds(off, _L)]
        zv = z_v[pl.ds(off, _L)]
        disc_v[pl.ds(off, _L)] = plsc.load_gather(cb_v, [mv])
        plsc.addupdate_scatter(histf_v, [laneoff + mv], ones16)

        x = (zv + 1.0) * (kmax / 2.0)
        x = jnp.minimum(jnp.maximum(x, 0.0), float(kmax))
        e0 = (x + 0.5).astype(jnp.int32)
        a = jnp.maximum(e0 - 1, 0)
        b = jnp.minimum(e0, kmax)
        c = jnp.minimum(e0 + 1, kmax)
        da = jnp.abs(zv - plsc.load_gather(cb_v, [a]))
        db = jnp.abs(zv - plsc.load_gather(cb_v, [b]))
        dc = jnp.abs(zv - plsc.load_gather(cb_v, [c]))
        bi = a
        bd = da
        upd = db < bd
        bi = jnp.where(upd, b, bi)
        bd = jnp.where(upd, db, bd)
        bi = jnp.where(dc < bd, c, bi)
        enc_v[pl.ds(off, _L)] = bi
        return carry

    lax.fori_loop(0, iters, body, 0)

    for cidx in range(kk // _L):
        acc = histf_v[pl.ds(cidx * _L, _L)]
        for l in range(1, _L):
            acc = acc + histf_v[pl.ds(l * kk + cidx * _L, _L)]
        histo_v[pl.ds(cidx * _L, _L)] = acc

    pltpu.sync_copy(disc_v, disc_hbm.at[pl.ds(base, chunk)])
    pltpu.sync_copy(enc_v, enc_hbm.at[pl.ds(base, chunk)])
    pltpu.sync_copy(histo_v, hist_hbm.at[wid])


def _perp_kernel(h_ref, p_ref, *, n_total):
    avg = jnp.sum(h_ref[...], axis=0) / jnp.float32(n_total)
    p_ref[...] = jnp.exp(-jnp.sum(avg * jnp.log(avg + 1e-10))).reshape(1, 1)


def kernel(z, codebook, log_temperature, gumbel_noise):
    B, T, D = z.shape
    K = codebook.shape[0]
    N = B * T * D
    NR = N // K
    R = 56
    nsteps = pl.cdiv(NR, R)

    z2 = z.reshape(NR, K)
    g3 = gumbel_noise.reshape(NR, K, K)
    cb2 = codebook.reshape(1, K)
    krev2 = jnp.arange(K - 1, -1, -1, dtype=jnp.float32).reshape(1, K)

    m = pl.pallas_call(
        _argmax_kernel,
        grid=(nsteps,),
        in_specs=[
            pl.BlockSpec((R, K), lambda b: (b, 0)),
            pl.BlockSpec((1, K), lambda b: (0, 0)),
            pl.BlockSpec((1, K), lambda b: (0, 0)),
            pl.BlockSpec((R, K, K), lambda b: (b, 0, 0)),
        ],
        out_specs=pl.BlockSpec((R, K), lambda b: (b, 0)),
        out_shape=jax.ShapeDtypeStruct((NR, K), jnp.int32),
    )(z2, cb2, krev2, g3)

    chunk = N // _NW
    disc, enc, hist = pl.kernel(
        functools.partial(_sc_kernel, chunk=chunk, iters=chunk // _L, kk=K),
        out_type=[
            jax.ShapeDtypeStruct((N,), jnp.float32),
            jax.ShapeDtypeStruct((N,), jnp.int32),
            jax.ShapeDtypeStruct((_NW, K), jnp.float32),
        ],
        mesh=plsc.VectorSubcoreMesh(core_axis_name="c", subcore_axis_name="s",
                                    num_cores=_NC, num_subcores=_NS),
        compiler_params=pltpu.CompilerParams(needs_layout_passes=False),
        scratch_types=[
            pltpu.VMEM((chunk,), jnp.int32),
            pltpu.VMEM((chunk,), jnp.float32),
            pltpu.VMEM((K,), jnp.float32),
            pltpu.VMEM((chunk,), jnp.float32),
            pltpu.VMEM((chunk,), jnp.int32),
            pltpu.VMEM((_L * K,), jnp.float32),
            pltpu.VMEM((K,), jnp.float32),
        ],
    )(m.reshape(N), z.reshape(N), codebook)

    perp = pl.pallas_call(
        functools.partial(_perp_kernel, n_total=N),
        in_specs=[pl.BlockSpec((_NW, K), lambda: (0, 0))],
        out_specs=pl.BlockSpec((1, 1), lambda: (0, 0)),
        out_shape=jax.ShapeDtypeStruct((1, 1), jnp.float32),
    )(hist)

    return (disc.reshape(B, T, D), perp[0, 0], enc)


# speedup vs baseline: 2.8260x; 1.0032x over previous
import functools

import jax
import jax.numpy as jnp
from jax import lax
from jax.experimental import pallas as pl
from jax.experimental.pallas import tpu as pltpu
from jax.experimental.pallas import tpu_sc as plsc

_NC, _NS, _L = 2, 16, 16
_NW = _NC * _NS


def _argmax_kernel(z_ref, cb_ref, kr_ref, g_ref, m_ref):
    zb = z_ref[...]
    cb3 = cb_ref[...].reshape(1, 1, -1)
    k = cb3.shape[-1]
    krev3 = kr_ref[...].reshape(1, 1, k)
    y = g_ref[...] - jnp.abs(zb[:, :, None] - cb3)
    v = jnp.max(y, axis=-1, keepdims=True)
    cand = jnp.where(y == v, krev3, jnp.float32(0.0))
    m_ref[...] = (jnp.float32(k - 1) - jnp.max(cand, axis=-1)
                  ).astype(jnp.int32)


def _sc_kernel(m_hbm, z_hbm, cb_hbm, disc_hbm, enc_hbm, hist_hbm,
               m_v, z_v, cb_v, disc_v, enc_v, histf_v, histo_v,
               *, chunk, iters, kk):
    wid = lax.axis_index("s") * _NC + lax.axis_index("c")
    base = wid * chunk
    pltpu.sync_copy(m_hbm.at[pl.ds(base, chunk)], m_v)
    pltpu.sync_copy(z_hbm.at[pl.ds(base, chunk)], z_v)
    pltpu.sync_copy(cb_hbm, cb_v)

    zeros16 = jnp.zeros((_L,), jnp.float32)
    for j in range(_L * kk // _L):
        histf_v[pl.ds(j * _L, _L)] = zeros16
    ones16 = jnp.ones((_L,), jnp.float32)
    laneoff = lax.broadcasted_iota(jnp.int32, (_L,), 0) * kk
    kmax = kk - 1

    def hbody(i, carry):
        mv = m_v[pl.ds(i * _L, _L)]
        plsc.addupdate_scatter(histf_v, [laneoff + mv], ones16)
        return carry

    lax.fori_loop(0, iters, hbody, 0)

    @plsc.parallel_loop(0, iters, unroll=4)
    def body(i):
        off = i * _L
        mv = m_v[pl.ds(off, _L)]
        zv = z_v[pl.ds(off, _L)]
        disc_v[pl.ds(off, _L)] = plsc.load_gather(cb_v, [mv])

        x = (zv + 1.0) * (kmax / 2.0)
        x = jnp.minimum(jnp.maximum(x, 0.0), float(kmax))
        e0 = (x + 0.5).astype(jnp.int32)
        a = jnp.maximum(e0 - 1, 0)
        b = jnp.minimum(e0, kmax)
        c = jnp.minimum(e0 + 1, kmax)
        da = jnp.abs(zv - plsc.load_gather(cb_v, [a]))
        db = jnp.abs(zv - plsc.load_gather(cb_v, [b]))
        dc = jnp.abs(zv - plsc.load_gather(cb_v, [c]))
        bi = a
        bd = da
        upd = db < bd
        bi = jnp.where(upd, b, bi)
        bd = jnp.where(upd, db, bd)
        bi = jnp.where(dc < bd, c, bi)
        enc_v[pl.ds(off, _L)] = bi

    for cidx in range(kk // _L):
        acc = histf_v[pl.ds(cidx * _L, _L)]
        for l in range(1, _L):
            acc = acc + histf_v[pl.ds(l * kk + cidx * _L, _L)]
        histo_v[pl.ds(cidx * _L, _L)] = acc

    pltpu.sync_copy(disc_v, disc_hbm.at[pl.ds(base, chunk)])
    pltpu.sync_copy(enc_v, enc_hbm.at[pl.ds(base, chunk)])
    pltpu.sync_copy(histo_v, hist_hbm.at[wid])


def _perp_kernel(h_ref, p_ref, *, n_total):
    avg = jnp.sum(h_ref[...], axis=0) / jnp.float32(n_total)
    p_ref[...] = jnp.exp(-jnp.sum(avg * jnp.log(avg + 1e-10))).reshape(1, 1)


def kernel(z, codebook, log_temperature, gumbel_noise):
    B, T, D = z.shape
    K = codebook.shape[0]
    N = B * T * D
    NR = N // K
    R = 56
    nsteps = pl.cdiv(NR, R)

    z2 = z.reshape(NR, K)
    g3 = gumbel_noise.reshape(NR, K, K)
    cb2 = codebook.reshape(1, K)
    krev2 = jnp.arange(K - 1, -1, -1, dtype=jnp.float32).reshape(1, K)

    m = pl.pallas_call(
        _argmax_kernel,
        grid=(nsteps,),
        in_specs=[
            pl.BlockSpec((R, K), lambda b: (b, 0)),
            pl.BlockSpec((1, K), lambda b: (0, 0)),
            pl.BlockSpec((1, K), lambda b: (0, 0)),
            pl.BlockSpec((R, K, K), lambda b: (b, 0, 0)),
        ],
        out_specs=pl.BlockSpec((R, K), lambda b: (b, 0)),
        out_shape=jax.ShapeDtypeStruct((NR, K), jnp.int32),
    )(z2, cb2, krev2, g3)

    chunk = N // _NW
    disc, enc, hist = pl.kernel(
        functools.partial(_sc_kernel, chunk=chunk, iters=chunk // _L, kk=K),
        out_type=[
            jax.ShapeDtypeStruct((N,), jnp.float32),
            jax.ShapeDtypeStruct((N,), jnp.int32),
            jax.ShapeDtypeStruct((_NW, K), jnp.float32),
        ],
        mesh=plsc.VectorSubcoreMesh(core_axis_name="c", subcore_axis_name="s",
                                    num_cores=_NC, num_subcores=_NS),
        compiler_params=pltpu.CompilerParams(needs_layout_passes=False),
        scratch_types=[
            pltpu.VMEM((chunk,), jnp.int32),
            pltpu.VMEM((chunk,), jnp.float32),
            pltpu.VMEM((K,), jnp.float32),
            pltpu.VMEM((chunk,), jnp.float32),
            pltpu.VMEM((chunk,), jnp.int32),
            pltpu.VMEM((_L * K,), jnp.float32),
            pltpu.VMEM((K,), jnp.float32),
        ],
    )(m.reshape(N), z.reshape(N), codebook)

    perp = pl.pallas_call(
        functools.partial(_perp_kernel, n_total=N),
        in_specs=[pl.BlockSpec((_NW, K), lambda: (0, 0))],
        out_specs=pl.BlockSpec((1, 1), lambda: (0, 0)),
        out_shape=jax.ShapeDtypeStruct((1, 1), jnp.float32),
    )(hist)

    return (disc.reshape(B, T, D), perp[0, 0], enc)
